# Initial kernel scaffold; baseline (speedup 1.0000x reference)
import functools

import jax
import jax.numpy as jnp
from jax import lax
from jax.experimental import pallas as pl
from jax.experimental.pallas import tpu as pltpu
from jax.experimental.pallas import tpu_sc as plsc

_VOCAB = 100000
_D = 64
_SEQ = 200
_BATCH = 4096
_NC = 2
_NS = 16
_NW = _NC * _NS
_RPW = _BATCH // _NW
_HALF = _SEQ // 2
_LANES = 16


def _emb_body(ids_hbm, tok_hbm, pos_hbm, out_hbm, idx_all, pos_v, rows_v, sem):
    wid = lax.axis_index("s") * _NC + lax.axis_index("c")
    base = wid * _RPW

    pltpu.sync_copy(pos_hbm.at[pl.ds(0, _SEQ)], pos_v)
    pltpu.sync_copy(ids_hbm.at[pl.ds(base, _RPW)], idx_all)

    def row_body(r, _):
        for h in range(2):
            pltpu.async_copy(tok_hbm.at[idx_all.at[r, h]], rows_v, sem).wait()

            @plsc.parallel_loop(0, _HALF, 1, unroll=4)
            def _add(i):
                for j in range(_D // _LANES):
                    v = pos_v[h * _HALF + i, pl.ds(j * _LANES, _LANES)]
                    plsc.addupdate(rows_v.at[i, pl.ds(j * _LANES, _LANES)], v)

            pltpu.sync_copy(rows_v, out_hbm.at[base + r, pl.ds(h * _HALF, _HALF)])
        return ()

    lax.fori_loop(0, _RPW, row_body, ())


@jax.jit
def _emb_call(ids3, token_embedding, position_embedding):
    mesh = plsc.VectorSubcoreMesh(
        core_axis_name="c", subcore_axis_name="s",
        num_cores=_NC, num_subcores=_NS,
    )
    fn = functools.partial(
        pl.kernel,
        out_type=jax.ShapeDtypeStruct((_BATCH, _SEQ, _D), jnp.float32),
        mesh=mesh,
        scratch_types=[
            pltpu.VMEM((_RPW, 2, _HALF), jnp.int32),
            pltpu.VMEM((_SEQ, _D), jnp.float32),
            pltpu.VMEM((_HALF, _D), jnp.float32),
            pltpu.SemaphoreType.DMA,
        ],
    )(_emb_body)
    return fn(ids3, token_embedding, position_embedding)


def kernel(input_ids, token_embedding, position_embedding):
    ids3 = input_ids.astype(jnp.int32).reshape(_BATCH, 2, _HALF)
    return _emb_call(ids3, token_embedding, position_embedding)


# speedup vs baseline: 2.9370x; 2.9370x over previous
import functools

import jax
import jax.numpy as jnp
from jax import lax
from jax.experimental import pallas as pl
from jax.experimental.pallas import tpu as pltpu
from jax.experimental.pallas import tpu_sc as plsc

_VOCAB = 100000
_D = 64
_SEQ = 200
_BATCH = 4096
_NC = 2
_NS = 16
_NW = _NC * _NS
_RPW = _BATCH // _NW
_HALF = _SEQ // 2
_LANES = 16


def _emb_body(ids_hbm, tok_hbm, pos_hbm, out_hbm, idx_all, pos_v, rows_v, sem):
    wid = lax.axis_index("s") * _NC + lax.axis_index("c")
    base = wid * _RPW

    pltpu.sync_copy(pos_hbm.at[pl.ds(0, _SEQ)], pos_v)
    pltpu.sync_copy(ids_hbm.at[pl.ds(base, _RPW)], idx_all)

    def row_body(r, _):
        for h in range(2):
            pltpu.async_copy(tok_hbm.at[idx_all.at[r, h]], rows_v, sem).wait()

            @plsc.parallel_loop(0, _HALF, 1, unroll=4)
            def _add(i):
                for j in range(_D // _LANES):
                    v = pos_v[h * _HALF + i, pl.ds(j * _LANES, _LANES)]
                    plsc.addupdate(rows_v.at[i, pl.ds(j * _LANES, _LANES)], v)

            pltpu.sync_copy(rows_v, out_hbm.at[base + r, pl.ds(h * _HALF, _HALF)])
        return ()

    lax.fori_loop(0, _RPW, row_body, ())


@jax.jit
def _emb_call(ids3, token_embedding, position_embedding):
    mesh = plsc.VectorSubcoreMesh(
        core_axis_name="c", subcore_axis_name="s",
        num_cores=_NC, num_subcores=_NS,
    )
    fn = functools.partial(
        pl.kernel,
        out_type=jax.ShapeDtypeStruct((_BATCH, _SEQ, _D), jnp.float32),
        mesh=mesh,
        scratch_types=[
            pltpu.VMEM((_RPW, 2, _HALF), jnp.int32),
            pltpu.VMEM((_SEQ, _D), jnp.float32),
            pltpu.VMEM((_HALF, _D), jnp.float32),
            pltpu.SemaphoreType.DMA,
        ],
        compiler_params=pltpu.CompilerParams(use_tc_tiling_on_sc=False),
    )(_emb_body)
    return fn(ids3, token_embedding, position_embedding)


def kernel(input_ids, token_embedding, position_embedding):
    ids3 = input_ids.astype(jnp.int32).reshape(_BATCH, 2, _HALF)
    return _emb_call(ids3, token_embedding, position_embedding)


# speedup vs baseline: 3.9326x; 1.3390x over previous
import functools

import jax
import jax.numpy as jnp
from jax import lax
from jax.experimental import pallas as pl
from jax.experimental.pallas import tpu as pltpu
from jax.experimental.pallas import tpu_sc as plsc

_VOCAB = 100000
_D = 64
_SEQ = 200
_BATCH = 4096
_NC = 2
_NS = 16
_NW = _NC * _NS
_HALF = _SEQ // 2
_CPW = _BATCH * 2 // _NW
_NB = 4
_LANES = 16


def _emb_body(ids_hbm, tok_hbm, pos_hbm, out_hbm, idx_all, pos_v, *bufs):
    rows = bufs[:_NB]
    gsem = bufs[_NB:2 * _NB]
    ssem = bufs[2 * _NB:3 * _NB]

    wid = lax.axis_index("s") * _NC + lax.axis_index("c")
    base = wid * _CPW

    pltpu.sync_copy(pos_hbm.at[pl.ds(0, _SEQ)], pos_v)
    pltpu.sync_copy(ids_hbm.at[pl.ds(base, _CPW)], idx_all)

    for b in range(_NB):
        pltpu.async_copy(tok_hbm.at[idx_all.at[b]], rows[b], gsem[b])

    def outer(k, _):
        g = k * _NB
        for b in range(_NB):
            c = g + b
            pltpu.make_async_copy(tok_hbm.at[idx_all.at[c]], rows[b], gsem[b]).wait()

            @plsc.parallel_loop(0, _HALF, 1, unroll=4)
            def _add(i):
                for j in range(_D // _LANES):
                    v = pos_v[(b % 2) * _HALF + i, pl.ds(j * _LANES, _LANES)]
                    plsc.addupdate(rows[b].at[i, pl.ds(j * _LANES, _LANES)], v)

            dst = out_hbm.at[base + c]
            pltpu.async_copy(rows[b], dst, ssem[b])
            pltpu.make_async_copy(rows[b], dst, ssem[b]).wait()

            @pl.when(c + _NB < _CPW)
            def _():
                pltpu.async_copy(tok_hbm.at[idx_all.at[c + _NB]], rows[b], gsem[b])
        return ()

    lax.fori_loop(0, _CPW // _NB, outer, ())


@jax.jit
def _emb_call(ids2, token_embedding, position_embedding):
    mesh = plsc.VectorSubcoreMesh(
        core_axis_name="c", subcore_axis_name="s",
        num_cores=_NC, num_subcores=_NS,
    )
    fn = functools.partial(
        pl.kernel,
        out_type=jax.ShapeDtypeStruct((_BATCH * 2, _HALF, _D), jnp.float32),
        mesh=mesh,
        scratch_types=(
            [pltpu.VMEM((_CPW, _HALF), jnp.int32),
             pltpu.VMEM((_SEQ, _D), jnp.float32)]
            + [pltpu.VMEM((_HALF, _D), jnp.float32)] * _NB
            + [pltpu.SemaphoreType.DMA] * (2 * _NB)
        ),
        compiler_params=pltpu.CompilerParams(use_tc_tiling_on_sc=False),
    )(_emb_body)
    return fn(ids2, token_embedding, position_embedding)


def kernel(input_ids, token_embedding, position_embedding):
    ids2 = input_ids.astype(jnp.int32).reshape(_BATCH * 2, _HALF)
    out = _emb_call(ids2, token_embedding, position_embedding)
    return out.reshape(_BATCH, _SEQ, _D)


# speedup vs baseline: 3.9442x; 1.0029x over previous
import functools

import jax
import jax.numpy as jnp
from jax import lax
from jax.experimental import pallas as pl
from jax.experimental.pallas import tpu as pltpu
from jax.experimental.pallas import tpu_sc as plsc

_VOCAB = 100000
_D = 64
_SEQ = 200
_BATCH = 4096
_NC = 2
_NS = 16
_NW = _NC * _NS
_RPW = _BATCH // _NW
_CHUNK = (104, 96)
_OFF = (0, 104)
_CMAX = 104
_CPW = 2 * _RPW
_NB = 4
_LANES = 16


def _emb_body(ids_hbm, tok_hbm, pos_hbm, out_hbm, idx_all, pos_v, *bufs):
    rows = bufs[:_NB]
    gsem = bufs[_NB:2 * _NB]
    ssem = bufs[2 * _NB:3 * _NB]

    wid = lax.axis_index("s") * _NC + lax.axis_index("c")
    base = wid * _RPW

    pltpu.sync_copy(pos_hbm.at[pl.ds(0, _SEQ)], pos_v)
    pltpu.sync_copy(ids_hbm.at[pl.ds(base, _RPW)], idx_all)

    def gather(r, h, b):
        n = _CHUNK[h]
        src = tok_hbm.at[idx_all.at[r, pl.ds(_OFF[h], n)]]
        return pltpu.make_async_copy(src, rows[b].at[pl.ds(0, n)], gsem[b])

    for b in range(_NB):
        gather(b // 2, b % 2, b).start()

    def outer(k, _):
        r0 = k * (_NB // 2)
        for b in range(_NB):
            r = r0 + b // 2
            h = b % 2
            n = _CHUNK[h]
            gather(r, h, b).wait()

            @plsc.parallel_loop(0, n, 1, unroll=4)
            def _add(i):
                for j in range(_D // _LANES):
                    v = pos_v[_OFF[h] + i, pl.ds(j * _LANES, _LANES)]
                    plsc.addupdate(rows[b].at[i, pl.ds(j * _LANES, _LANES)], v)

            dst = out_hbm.at[base + r, pl.ds(_OFF[h], n)]
            scat = pltpu.make_async_copy(rows[b].at[pl.ds(0, n)], dst, ssem[b])
            scat.start()
            scat.wait()

            @pl.when(2 * r + h + _NB < _CPW)
            def _():
                gather(r + _NB // 2, h, b).start()
        return ()

    lax.fori_loop(0, _CPW // _NB, outer, ())


@jax.jit
def _emb_call(ids, token_embedding, position_embedding):
    mesh = plsc.VectorSubcoreMesh(
        core_axis_name="c", subcore_axis_name="s",
        num_cores=_NC, num_subcores=_NS,
    )
    fn = functools.partial(
        pl.kernel,
        out_type=jax.ShapeDtypeStruct((_BATCH, _SEQ, _D), jnp.float32),
        mesh=mesh,
        scratch_types=(
            [pltpu.VMEM((_RPW, _SEQ), jnp.int32),
             pltpu.VMEM((_SEQ, _D), jnp.float32)]
            + [pltpu.VMEM((_CMAX, _D), jnp.float32)] * _NB
            + [pltpu.SemaphoreType.DMA] * (2 * _NB)
        ),
        compiler_params=pltpu.CompilerParams(use_tc_tiling_on_sc=False),
    )(_emb_body)
    return fn(ids, token_embedding, position_embedding)


def kernel(input_ids, token_embedding, position_embedding):
    return _emb_call(input_ids.astype(jnp.int32), token_embedding,
                     position_embedding)


# speedup vs baseline: 6.6490x; 1.6858x over previous
import functools

import jax
import jax.numpy as jnp
from jax import lax
from jax.experimental import pallas as pl
from jax.experimental.pallas import tpu as pltpu
from jax.experimental.pallas import tpu_sc as plsc

_VOCAB = 100000
_D = 64
_SEQ = 200
_BATCH = 4096
_NC = 2
_NS = 16
_NW = _NC * _NS
_RPW = _BATCH // _NW
_CHUNK = (104, 96)
_OFF = (0, 104)
_CMAX = 104
_CPW = 2 * _RPW
_NB = 4
_LANES = 16


def _emb_body(ids_hbm, tok_hbm, pos_hbm, out_hbm, idx_all, pos_v, *bufs):
    rows = bufs[:_NB]
    gsem = bufs[_NB:2 * _NB]
    ssem = bufs[2 * _NB:3 * _NB]

    wid = lax.axis_index("s") * _NC + lax.axis_index("c")
    base = wid * _RPW

    pltpu.sync_copy(pos_hbm.at[pl.ds(0, _SEQ)], pos_v)
    pltpu.sync_copy(ids_hbm.at[pl.ds(base, _RPW)], idx_all)

    def gather(r, h, b):
        n = _CHUNK[h]
        src = tok_hbm.at[idx_all.at[r, pl.ds(_OFF[h], n)]]
        return pltpu.make_async_copy(src, rows[b].at[pl.ds(0, n)], gsem[b])

    for b in range(_NB):
        gather(b // 2, b % 2, b).start()

    def outer(k, _):
        r0 = k * (_NB // 2)
        for b in range(_NB):
            r = r0 + b // 2
            h = b % 2
            n = _CHUNK[h]
            gather(r, h, b).wait()

            @plsc.parallel_loop(0, n, 1, unroll=4)
            def _add(i):
                for j in range(_D // _LANES):
                    v = pos_v[_OFF[h] + i, pl.ds(j * _LANES, _LANES)]
                    plsc.addupdate(rows[b].at[i, pl.ds(j * _LANES, _LANES)], v)

            dst = out_hbm.at[base + r, pl.ds(_OFF[h], n), pl.ds(0, _D)]
            scat = pltpu.make_async_copy(rows[b].at[pl.ds(0, n)], dst, ssem[b])
            scat.start()
            scat.wait()

            @pl.when(2 * r + h + _NB < _CPW)
            def _():
                gather(r + _NB // 2, h, b).start()
        return ()

    lax.fori_loop(0, _CPW // _NB, outer, ())


@jax.jit
def _emb_call(ids, token_embedding, position_embedding):
    mesh = plsc.VectorSubcoreMesh(
        core_axis_name="c", subcore_axis_name="s",
        num_cores=_NC, num_subcores=_NS,
    )
    fn = functools.partial(
        pl.kernel,
        out_type=jax.ShapeDtypeStruct((_BATCH, _SEQ, 2 * _D), jnp.float32),
        mesh=mesh,
        scratch_types=(
            [pltpu.VMEM((_RPW, _SEQ), jnp.int32),
             pltpu.VMEM((_SEQ, _D), jnp.float32)]
            + [pltpu.VMEM((_CMAX, _D), jnp.float32)] * _NB
            + [pltpu.SemaphoreType.DMA] * (2 * _NB)
        ),
        compiler_params=pltpu.CompilerParams(use_tc_tiling_on_sc=False),
    )(_emb_body)
    return fn(ids, token_embedding, position_embedding)


def kernel(input_ids, token_embedding, position_embedding):
    out128 = _emb_call(input_ids.astype(jnp.int32), token_embedding,
                       position_embedding)
    return out128[:, :, :_D]


# speedup vs baseline: 7.4437x; 1.1195x over previous
import functools

import jax
import jax.numpy as jnp
from jax import lax
from jax.experimental import pallas as pl
from jax.experimental.pallas import tpu as pltpu
from jax.experimental.pallas import tpu_sc as plsc

_VOCAB = 100000
_D = 64
_SEQ = 200
_BATCH = 4096
_NC = 2
_NS = 16
_NW = _NC * _NS
_RPW = _BATCH // _NW
_CHUNK = (104, 96)
_OFF = (0, 104)
_CMAX = 104
_CPW = 2 * _RPW
_NS_RING = 8
_PF = 4
_LANES = 16


def _emb_body(ids_hbm, tok_hbm, pos_hbm, out_hbm, idx_all, pos_v, *bufs):
    rows = bufs[:_NS_RING]
    gsem = bufs[_NS_RING:2 * _NS_RING]
    ssem = bufs[2 * _NS_RING:3 * _NS_RING]

    wid = lax.axis_index("s") * _NC + lax.axis_index("c")
    base = wid * _RPW

    pltpu.sync_copy(pos_hbm.at[pl.ds(0, _SEQ)], pos_v)
    pltpu.sync_copy(ids_hbm.at[pl.ds(base, _RPW)], idx_all)

    def gather(r, h, b):
        n = _CHUNK[h]
        src = tok_hbm.at[idx_all.at[r, pl.ds(_OFF[h], n)]]
        return pltpu.make_async_copy(src, rows[b].at[pl.ds(0, n)], gsem[b])

    def scatter(r, h, b):
        n = _CHUNK[h]
        dst = out_hbm.at[base + r, pl.ds(_OFF[h], n), pl.ds(0, _D)]
        return pltpu.make_async_copy(rows[b].at[pl.ds(0, n)], dst, ssem[b])

    for b in range(_PF):
        gather(b // 2, b % 2, b).start()

    def outer(k, _):
        c0 = k * _NS_RING
        for b in range(_NS_RING):
            r = c0 // 2 + b // 2
            h = b % 2
            gather(r, h, b).wait()

            n = _CHUNK[h]

            @plsc.parallel_loop(0, n, 1, unroll=4)
            def _add(i):
                for j in range(_D // _LANES):
                    v = pos_v[_OFF[h] + i, pl.ds(j * _LANES, _LANES)]
                    plsc.addupdate(rows[b].at[i, pl.ds(j * _LANES, _LANES)], v)

            scatter(r, h, b).start()

            b4 = (b + _PF) % _NS_RING
            cm = c0 + b - _PF
            cp = c0 + b + _PF
            rm = cm // 2
            rp = cp // 2

            @pl.when(cm >= 0)
            def _():
                scatter(rm, h, b4).wait()

            @pl.when(cp < _CPW)
            def _():
                gather(rp, h, b4).start()
        return ()

    lax.fori_loop(0, _CPW // _NS_RING, outer, ())

    for b in range(_PF):
        c = _CPW - _PF + b
        scatter(c // 2, c % 2, c % _NS_RING).wait()


@jax.jit
def _emb_call(ids, token_embedding, position_embedding):
    mesh = plsc.VectorSubcoreMesh(
        core_axis_name="c", subcore_axis_name="s",
        num_cores=_NC, num_subcores=_NS,
    )
    fn = functools.partial(
        pl.kernel,
        out_type=jax.ShapeDtypeStruct((_BATCH, _SEQ, 2 * _D), jnp.float32),
        mesh=mesh,
        scratch_types=(
            [pltpu.VMEM((_RPW, _SEQ), jnp.int32),
             pltpu.VMEM((_SEQ, _D), jnp.float32)]
            + [pltpu.VMEM((_CMAX, _D), jnp.float32)] * _NS_RING
            + [pltpu.SemaphoreType.DMA] * (2 * _NS_RING)
        ),
        compiler_params=pltpu.CompilerParams(use_tc_tiling_on_sc=False),
    )(_emb_body)
    return fn(ids, token_embedding, position_embedding)


def kernel(input_ids, token_embedding, position_embedding):
    out128 = _emb_call(input_ids.astype(jnp.int32), token_embedding,
                       position_embedding)
    return out128[:, :, :_D]
